# baseline (speedup 1.0000x reference)
import functools
import math

import jax
import jax.numpy as jnp
from jax import lax
from jax.experimental import pallas as pl
from jax.experimental.pallas import tpu as pltpu
from jax.experimental.pallas import tpu_sc as plsc

D_MODEL_K = 64
VOCAB_K = 1000000
SCALE_K = math.sqrt(D_MODEL_K)

NC = 2
NS = 16
NW = NC * NS
CHUNK = 128


def _emb_body(x_hbm, lut_hbm, out_hbm, idx_v, rows_v, sem):
    wid = lax.axis_index("s") * NC + lax.axis_index("c")
    n_total = x_hbm.shape[0]
    per_w = n_total // NW
    n_chunks = per_w // CHUNK
    base = wid * per_w

    def chunk_body(g, carry):
        row0 = base + g * CHUNK
        pltpu.sync_copy(x_hbm.at[pl.ds(row0, CHUNK)], idx_v)
        pltpu.async_copy(lut_hbm.at[idx_v], rows_v, sem).wait()

        def scale_row(r, c2):
            for c in range(D_MODEL_K // 16):
                sl = (r, pl.ds(c * 16, 16))
                rows_v[sl] = rows_v[sl] * SCALE_K
            return c2

        lax.fori_loop(0, CHUNK, scale_row, 0, unroll=2)
        pltpu.sync_copy(rows_v, out_hbm.at[pl.ds(row0, CHUNK)])
        return carry

    lax.fori_loop(0, n_chunks, chunk_body, 0)


@jax.jit
def _emb_call(x_flat, lut):
    n = x_flat.shape[0]
    mesh = plsc.VectorSubcoreMesh(core_axis_name="c", subcore_axis_name="s")
    fn = functools.partial(
        pl.kernel,
        out_type=jax.ShapeDtypeStruct((n, D_MODEL_K), jnp.float32),
        mesh=mesh,
        scratch_types=[
            pltpu.VMEM((CHUNK,), jnp.int32),
            pltpu.VMEM((CHUNK, D_MODEL_K), jnp.float32),
            pltpu.SemaphoreType.DMA,
        ],
        compiler_params=pltpu.CompilerParams(use_tc_tiling_on_sc=False),
    )(_emb_body)
    return fn(x_flat, lut)


def kernel(x, lut):
    b, s = x.shape
    x_flat = x.reshape(b * s).astype(jnp.int32)
    out = _emb_call(x_flat, lut)
    return out.reshape(b, s, D_MODEL_K)
